# baseline (speedup 1.0000x reference)
import functools

import jax
import jax.numpy as jnp
from jax.experimental import pallas as pl
from jax.experimental.pallas import tpu as pltpu

E = 8
K = 2
BLK = 256
RBLK = 256
NEG = -1e30


def _ln_router_kernel(x_ref, lnw_ref, lnb_ref, wg_ref, xn_ref, logits_ref):
    x = x_ref[...]
    mu = jnp.mean(x, axis=-1, keepdims=True)
    var = jnp.mean((x - mu) ** 2, axis=-1, keepdims=True)
    xn = (x - mu) / jnp.sqrt(var + 1e-6) * lnw_ref[...] + lnb_ref[...]
    xn_ref[...] = xn
    logits_ref[...] = jnp.dot(xn, wg_ref[...],
                              preferred_element_type=jnp.float32)


def _route_kernel(logits_ref, topi_ref, gates_ref, dest_ref, cnt_ref,
                  aux_ref, bal_ref):
    logits = logits_ref[...]
    T = logits.shape[0]
    iota = jax.lax.broadcasted_iota(jnp.int32, (T, E), 1)
    m1 = jnp.max(logits, axis=-1, keepdims=True)
    i1 = jnp.min(jnp.where(logits == m1, iota, E), axis=-1)
    masked = jnp.where(iota == i1[:, None], NEG, logits)
    m2 = jnp.max(masked, axis=-1, keepdims=True)
    i2 = jnp.min(jnp.where(masked == m2, iota, E), axis=-1)
    topi_ref[...] = jnp.stack([i1, i2], axis=-1)
    e2 = jnp.exp(m2 - m1)[:, 0]
    g1 = 1.0 / (1.0 + e2)
    gates_ref[...] = jnp.stack([g1, 1.0 - g1], axis=-1)
    iota_e = jax.lax.broadcasted_iota(jnp.int32, (E, T), 0)
    oh1 = (i1[None, :] == iota_e).astype(jnp.int32)
    oh2 = (i2[None, :] == iota_e).astype(jnp.int32)
    both = oh1 + oh2
    cum = both
    shift = 1
    while shift < T:
        cum = cum + jnp.concatenate(
            [jnp.zeros((E, shift), jnp.int32), cum[:, :-shift]], axis=1)
        shift *= 2
    counts = cum[:, -1:]
    acc = counts
    shift = 1
    while shift < E:
        acc = acc + jnp.concatenate(
            [jnp.zeros((shift, 1), jnp.int32), acc[:-shift]], axis=0)
        shift *= 2
    off = acc - counts
    excl = cum - both
    d1 = jnp.sum(oh1 * (off + excl), axis=0)
    d2 = jnp.sum(oh2 * (off + excl + oh1), axis=0)
    dest_ref[...] = jnp.stack([d1, d2], axis=-1)
    cnt_ref[...] = jnp.concatenate(
        [jnp.transpose(counts), jnp.transpose(off)], axis=0)
    ex = jnp.exp(logits - m1)
    sumex = jnp.sum(ex, axis=-1, keepdims=True)
    probs = ex / sumex
    P = jnp.mean(probs, axis=0)
    dens = jnp.mean(both.astype(jnp.float32), axis=1)
    aux_ref[...] = (E * jnp.sum(dens * P)).reshape(1, 1)
    z = m1[:, 0] + jnp.log(sumex[:, 0])
    bal_ref[...] = jnp.mean(z * z).reshape(1, 1)


def _expert_kernel(te_ref, rb_ref, lo_ref, hi_ref, tv_ref,
                   xg_ref, g_ref, w1_ref, b1_ref, w2_ref, b2_ref,
                   out_ref):
    s = pl.program_id(0)

    @pl.when(tv_ref[s] > 0)
    def _():
        h = jnp.dot(xg_ref[...], w1_ref[0], preferred_element_type=jnp.float32)
        h = jax.nn.gelu(h + b1_ref[0])
        y = jnp.dot(h, w2_ref[0], preferred_element_type=jnp.float32)
        y = (y + b2_ref[0]) * g_ref[0]
        row = rb_ref[s] * BLK + jax.lax.broadcasted_iota(
            jnp.int32, (BLK, 1), 0)
        mask = (row >= lo_ref[s]) & (row < hi_ref[s])
        out_ref[...] = jnp.where(mask, y, out_ref[...])


@functools.partial(jax.jit, static_argnames=("interpret",))
def kernel(x_img, ln_w, ln_b, Wg, W1, b1, W2, b2, interpret=False):
    Bb, S, C = x_img.shape
    T = Bb * S
    H = W2.shape[-1]
    TK = T * K
    NB = TK // BLK
    NT = NB + E - 1
    x = x_img.reshape(T, C)

    xn, logits = pl.pallas_call(
        _ln_router_kernel,
        grid=(T // RBLK,),
        in_specs=[
            pl.BlockSpec((RBLK, C), lambda i: (i, 0)),
            pl.BlockSpec((1, C), lambda i: (0, 0)),
            pl.BlockSpec((1, C), lambda i: (0, 0)),
            pl.BlockSpec((C, E), lambda i: (0, 0)),
        ],
        out_specs=[
            pl.BlockSpec((RBLK, C), lambda i: (i, 0)),
            pl.BlockSpec((RBLK, E), lambda i: (i, 0)),
        ],
        out_shape=[
            jax.ShapeDtypeStruct((T, C), jnp.float32),
            jax.ShapeDtypeStruct((T, E), jnp.float32),
        ],
        interpret=interpret,
    )(x, ln_w.reshape(1, C), ln_b.reshape(1, C), Wg)

    topi, gates, dest, cnt, aux, bal = pl.pallas_call(
        _route_kernel,
        out_shape=[
            jax.ShapeDtypeStruct((T, K), jnp.int32),
            jax.ShapeDtypeStruct((T, K), jnp.float32),
            jax.ShapeDtypeStruct((T, K), jnp.int32),
            jax.ShapeDtypeStruct((2, E), jnp.int32),
            jax.ShapeDtypeStruct((1, 1), jnp.float32),
            jax.ShapeDtypeStruct((1, 1), jnp.float32),
        ],
        interpret=interpret,
    )(logits)

    dest_flat = dest.reshape(TK)
    _, srow, g_s = jax.lax.sort(
        (dest_flat, jnp.arange(TK, dtype=jnp.int32), gates.reshape(TK)),
        num_keys=1)
    counts = cnt[0]
    off = cnt[1]
    end = off + counts
    first_b = off // BLK
    nb = jnp.where(counts > 0, jnp.maximum(end - 1, 0) // BLK - first_b + 1, 0)
    cum_nb = jnp.cumsum(nb)
    slot_start = cum_nb - nb
    sidx = jnp.arange(NT, dtype=jnp.int32)
    te = jnp.searchsorted(cum_nb, sidx, side="right").astype(jnp.int32)
    tv = (sidx < cum_nb[-1]).astype(jnp.int32)
    te = jnp.minimum(te, E - 1)
    rb = first_b[te] + (sidx - slot_start[te])
    rb = jnp.where(tv > 0, rb, NB - 1).astype(jnp.int32)
    lo = jnp.maximum(rb * BLK, off[te]).astype(jnp.int32)
    hi = jnp.minimum((rb + 1) * BLK, end[te]).astype(jnp.int32)

    xg = jnp.take(xn, srow // K, axis=0)
    g_col = g_s.reshape(NB, BLK, 1)

    grid_spec = pltpu.PrefetchScalarGridSpec(
        num_scalar_prefetch=5,
        grid=(NT,),
        in_specs=[
            pl.BlockSpec((BLK, C), lambda s, te, rb, lo, hi, tv: (rb[s], 0)),
            pl.BlockSpec((1, BLK, 1), lambda s, te, rb, lo, hi, tv: (rb[s], 0, 0)),
            pl.BlockSpec((1, C, H), lambda s, te, rb, lo, hi, tv: (te[s], 0, 0)),
            pl.BlockSpec((1, 1, H), lambda s, te, rb, lo, hi, tv: (te[s], 0, 0)),
            pl.BlockSpec((1, H, H), lambda s, te, rb, lo, hi, tv: (te[s], 0, 0)),
            pl.BlockSpec((1, 1, H), lambda s, te, rb, lo, hi, tv: (te[s], 0, 0)),
        ],
        out_specs=pl.BlockSpec((BLK, H), lambda s, te, rb, lo, hi, tv: (rb[s], 0)),
    )
    y_s = pl.pallas_call(
        _expert_kernel,
        grid_spec=grid_spec,
        out_shape=jax.ShapeDtypeStruct((TK, H), jnp.float32),
        interpret=interpret,
    )(te, rb, lo, hi, tv, xg, g_col, W1, b1.reshape(E, 1, H), W2,
      b2.reshape(E, 1, H))

    y_tok = jnp.take(y_s, dest_flat, axis=0)
    out = jnp.sum(y_tok.reshape(T, K, H), axis=1)

    results = out.reshape(Bb, S, H)
    id_experts = topi.reshape(Bb, S, K)
    return results, aux[0, 0], id_experts, bal[0, 0]


# speedup vs baseline: 1.6527x; 1.6527x over previous
import functools

import jax
import jax.numpy as jnp
from jax.experimental import pallas as pl
from jax.experimental.pallas import tpu as pltpu

E = 8
K = 2
BLK = 256
RBLK = 256
NEG = -1e30


def _ln_router_kernel(x_ref, lnw_ref, lnb_ref, wg_ref, xn_ref, logits_ref):
    x = x_ref[...]
    mu = jnp.mean(x, axis=-1, keepdims=True)
    var = jnp.mean((x - mu) ** 2, axis=-1, keepdims=True)
    xn = (x - mu) / jnp.sqrt(var + 1e-6) * lnw_ref[...] + lnb_ref[...]
    xn_ref[...] = xn
    logits_ref[...] = jnp.dot(xn, wg_ref[...],
                              preferred_element_type=jnp.float32)


def _route_kernel(logits_ref, topi_ref, gates_ref, key_ref, cnt_ref,
                  aux_ref, bal_ref):
    lt = jnp.transpose(logits_ref[...])
    T = lt.shape[1]
    TK = T * K
    iota_e = jax.lax.broadcasted_iota(jnp.int32, (E, T), 0)
    m1 = jnp.max(lt, axis=0, keepdims=True)
    i1 = jnp.min(jnp.where(lt == m1, iota_e, E), axis=0, keepdims=True)
    masked = jnp.where(iota_e == i1, NEG, lt)
    m2 = jnp.max(masked, axis=0, keepdims=True)
    i2 = jnp.min(jnp.where(masked == m2, iota_e, E), axis=0, keepdims=True)
    topi_ref[...] = jnp.concatenate([i1, i2], axis=0)
    g1 = 1.0 / (1.0 + jnp.exp(m2 - m1))
    gates_ref[...] = jnp.concatenate([g1, 1.0 - g1], axis=0)
    iota_t = jax.lax.broadcasted_iota(jnp.int32, (1, T), 1)
    key_ref[...] = jnp.concatenate(
        [i1 * TK + iota_t * K, i2 * TK + iota_t * K + 1], axis=0)
    both = ((i1 == iota_e).astype(jnp.int32)
            + (i2 == iota_e).astype(jnp.int32))
    counts = jnp.sum(both, axis=1, keepdims=True)
    acc = counts
    shift = 1
    while shift < E:
        acc = acc + jnp.concatenate(
            [jnp.zeros((shift, 1), jnp.int32), acc[:-shift]], axis=0)
        shift *= 2
    off = acc - counts
    cnt_ref[...] = jnp.concatenate(
        [jnp.transpose(counts), jnp.transpose(off)], axis=0)
    ex = jnp.exp(lt - m1)
    sumex = jnp.sum(ex, axis=0, keepdims=True)
    P = jnp.mean(ex / sumex, axis=1, keepdims=True)
    dens = counts.astype(jnp.float32) / T
    aux_ref[...] = (E * jnp.sum(dens * P)).reshape(1, 1)
    z = m1 + jnp.log(sumex)
    bal_ref[...] = jnp.mean(z * z).reshape(1, 1)


def _expert_kernel(te_ref, rb_ref, lo_ref, hi_ref, tv_ref,
                   xg_ref, w1_ref, b1_ref, w2_ref, b2_ref, out_ref):
    s = pl.program_id(0)

    @pl.when(tv_ref[s] > 0)
    def _():
        h = jnp.dot(xg_ref[...], w1_ref[0], preferred_element_type=jnp.float32)
        h = jax.nn.gelu(h + b1_ref[0])
        y = jnp.dot(h, w2_ref[0], preferred_element_type=jnp.float32)
        y = y + b2_ref[0]
        row = rb_ref[s] * BLK + jax.lax.broadcasted_iota(
            jnp.int32, (BLK, 1), 0)
        mask = (row >= lo_ref[s]) & (row < hi_ref[s])
        out_ref[...] = jnp.where(mask, y, out_ref[...])


@functools.partial(jax.jit, static_argnames=("interpret",))
def kernel(x_img, ln_w, ln_b, Wg, W1, b1, W2, b2, interpret=False):
    Bb, S, C = x_img.shape
    T = Bb * S
    H = W2.shape[-1]
    TK = T * K
    NB = TK // BLK
    NT = NB + E - 1
    x = x_img.reshape(T, C)

    xn, logits = pl.pallas_call(
        _ln_router_kernel,
        grid=(T // RBLK,),
        in_specs=[
            pl.BlockSpec((RBLK, C), lambda i: (i, 0)),
            pl.BlockSpec((1, C), lambda i: (0, 0)),
            pl.BlockSpec((1, C), lambda i: (0, 0)),
            pl.BlockSpec((C, E), lambda i: (0, 0)),
        ],
        out_specs=[
            pl.BlockSpec((RBLK, C), lambda i: (i, 0)),
            pl.BlockSpec((RBLK, E), lambda i: (i, 0)),
        ],
        out_shape=[
            jax.ShapeDtypeStruct((T, C), jnp.float32),
            jax.ShapeDtypeStruct((T, E), jnp.float32),
        ],
        interpret=interpret,
    )(x, ln_w.reshape(1, C), ln_b.reshape(1, C), Wg)

    topi_t, gates_t, key_t, cnt, aux, bal = pl.pallas_call(
        _route_kernel,
        out_shape=[
            jax.ShapeDtypeStruct((K, T), jnp.int32),
            jax.ShapeDtypeStruct((K, T), jnp.float32),
            jax.ShapeDtypeStruct((K, T), jnp.int32),
            jax.ShapeDtypeStruct((2, E), jnp.int32),
            jax.ShapeDtypeStruct((1, 1), jnp.float32),
            jax.ShapeDtypeStruct((1, 1), jnp.float32),
        ],
        interpret=interpret,
    )(logits)

    ks, = jax.lax.sort((key_t.reshape(TK),), num_keys=1)
    srow = ks % TK
    tok_s = srow // K
    _, dest = jax.lax.sort((srow, jnp.arange(TK, dtype=jnp.int32)),
                           num_keys=1)
    dest2 = dest.reshape(T, K)

    counts = cnt[0]
    off = cnt[1]
    end = off + counts
    first_b = off // BLK
    nb = jnp.where(counts > 0, jnp.maximum(end - 1, 0) // BLK - first_b + 1, 0)
    cum_nb = jnp.cumsum(nb)
    slot_start = cum_nb - nb
    sidx = jnp.arange(NT, dtype=jnp.int32)
    te = jnp.searchsorted(cum_nb, sidx, side="right").astype(jnp.int32)
    tv = (sidx < cum_nb[-1]).astype(jnp.int32)
    te = jnp.minimum(te, E - 1)
    rb = first_b[te] + (sidx - slot_start[te])
    rb = jnp.where(tv > 0, rb, NB - 1).astype(jnp.int32)
    lo = jnp.maximum(rb * BLK, off[te]).astype(jnp.int32)
    hi = jnp.minimum((rb + 1) * BLK, end[te]).astype(jnp.int32)

    xg = jnp.take(xn, tok_s, axis=0)

    grid_spec = pltpu.PrefetchScalarGridSpec(
        num_scalar_prefetch=5,
        grid=(NT,),
        in_specs=[
            pl.BlockSpec((BLK, C), lambda s, te, rb, lo, hi, tv: (rb[s], 0)),
            pl.BlockSpec((1, C, H), lambda s, te, rb, lo, hi, tv: (te[s], 0, 0)),
            pl.BlockSpec((1, 1, H), lambda s, te, rb, lo, hi, tv: (te[s], 0, 0)),
            pl.BlockSpec((1, H, H), lambda s, te, rb, lo, hi, tv: (te[s], 0, 0)),
            pl.BlockSpec((1, 1, H), lambda s, te, rb, lo, hi, tv: (te[s], 0, 0)),
        ],
        out_specs=pl.BlockSpec((BLK, H), lambda s, te, rb, lo, hi, tv: (rb[s], 0)),
    )
    y_s = pl.pallas_call(
        _expert_kernel,
        grid_spec=grid_spec,
        out_shape=jax.ShapeDtypeStruct((TK, H), jnp.float32),
        interpret=interpret,
    )(te, rb, lo, hi, tv, xg, W1, b1.reshape(E, 1, H), W2, b2.reshape(E, 1, H))

    g0 = gates_t[0].reshape(T, 1)
    g1 = gates_t[1].reshape(T, 1)
    out = (g0 * jnp.take(y_s, dest2[:, 0], axis=0)
           + g1 * jnp.take(y_s, dest2[:, 1], axis=0))

    results = out.reshape(Bb, S, H)
    id_experts = jnp.transpose(topi_t).reshape(Bb, S, K)
    return results, aux[0, 0], id_experts, bal[0, 0]


# speedup vs baseline: 1.6924x; 1.0240x over previous
import functools

import jax
import jax.numpy as jnp
from jax.experimental import pallas as pl
from jax.experimental.pallas import tpu as pltpu

E = 8
K = 2
BLK = 256
RBLK = 256
NEG = -1e30


def _ln_router_kernel(x_ref, lnw_ref, lnb_ref, wg_ref, xb_ref, logits_ref):
    x = x_ref[...]
    mu = jnp.mean(x, axis=-1, keepdims=True)
    var = jnp.mean((x - mu) ** 2, axis=-1, keepdims=True)
    xn = (x - mu) / jnp.sqrt(var + 1e-6) * lnw_ref[...] + lnb_ref[...]
    xb_ref[...] = xn.astype(jnp.bfloat16)
    logits_ref[...] = jnp.dot(xn, wg_ref[...],
                              preferred_element_type=jnp.float32)


def _route_kernel(logits_ref, topi_ref, gates_ref, key_ref, cnt_ref,
                  aux_ref, bal_ref):
    lt = jnp.transpose(logits_ref[...])
    T = lt.shape[1]
    TK = T * K
    iota_e = jax.lax.broadcasted_iota(jnp.int32, (E, T), 0)
    m1 = jnp.max(lt, axis=0, keepdims=True)
    i1 = jnp.min(jnp.where(lt == m1, iota_e, E), axis=0, keepdims=True)
    masked = jnp.where(iota_e == i1, NEG, lt)
    m2 = jnp.max(masked, axis=0, keepdims=True)
    i2 = jnp.min(jnp.where(masked == m2, iota_e, E), axis=0, keepdims=True)
    topi_ref[...] = jnp.concatenate([i1, i2], axis=0)
    g1 = 1.0 / (1.0 + jnp.exp(m2 - m1))
    gates_ref[...] = jnp.concatenate([g1, 1.0 - g1], axis=0)
    iota_t = jax.lax.broadcasted_iota(jnp.int32, (1, T), 1)
    key_ref[...] = jnp.concatenate(
        [i1 * TK + iota_t * K, i2 * TK + iota_t * K + 1], axis=0)
    both = ((i1 == iota_e).astype(jnp.int32)
            + (i2 == iota_e).astype(jnp.int32))
    counts = jnp.sum(both, axis=1, keepdims=True)
    acc = counts
    shift = 1
    while shift < E:
        acc = acc + jnp.concatenate(
            [jnp.zeros((shift, 1), jnp.int32), acc[:-shift]], axis=0)
        shift *= 2
    off = acc - counts
    cnt_ref[...] = jnp.concatenate(
        [jnp.transpose(counts), jnp.transpose(off)], axis=0)
    ex = jnp.exp(lt - m1)
    sumex = jnp.sum(ex, axis=0, keepdims=True)
    P = jnp.mean(ex / sumex, axis=1, keepdims=True)
    dens = counts.astype(jnp.float32) / T
    aux_ref[...] = (E * jnp.sum(dens * P)).reshape(1, 1)
    z = m1 + jnp.log(sumex)
    bal_ref[...] = jnp.mean(z * z).reshape(1, 1)


def _expert_kernel(te_ref, rb_ref, lo_ref, hi_ref, tv_ref,
                   xg_ref, g_ref, w1_ref, b1_ref, w2_ref, b2_ref, out_ref):
    s = pl.program_id(0)

    @pl.when(tv_ref[s] > 0)
    def _():
        w1 = w1_ref[0].astype(jnp.bfloat16)
        h = jnp.dot(xg_ref[...], w1, preferred_element_type=jnp.float32)
        h = jax.nn.gelu(h + b1_ref[0])
        w2 = w2_ref[0].astype(jnp.bfloat16)
        y = jnp.dot(h.astype(jnp.bfloat16), w2,
                    preferred_element_type=jnp.float32)
        y = (y + b2_ref[0]) * g_ref[0]
        row = rb_ref[s] * BLK + jax.lax.broadcasted_iota(
            jnp.int32, (BLK, 1), 0)
        mask = (row >= lo_ref[s]) & (row < hi_ref[s])
        out_ref[...] = jnp.where(mask, y, out_ref[...])


@functools.partial(jax.jit, static_argnames=("interpret",))
def kernel(x_img, ln_w, ln_b, Wg, W1, b1, W2, b2, interpret=False):
    Bb, S, C = x_img.shape
    T = Bb * S
    H = W2.shape[-1]
    TK = T * K
    NB = TK // BLK
    NT = NB + E - 1
    x = x_img.reshape(T, C)

    xb, logits = pl.pallas_call(
        _ln_router_kernel,
        grid=(T // RBLK,),
        in_specs=[
            pl.BlockSpec((RBLK, C), lambda i: (i, 0)),
            pl.BlockSpec((1, C), lambda i: (0, 0)),
            pl.BlockSpec((1, C), lambda i: (0, 0)),
            pl.BlockSpec((C, E), lambda i: (0, 0)),
        ],
        out_specs=[
            pl.BlockSpec((RBLK, C), lambda i: (i, 0)),
            pl.BlockSpec((RBLK, E), lambda i: (i, 0)),
        ],
        out_shape=[
            jax.ShapeDtypeStruct((T, C), jnp.bfloat16),
            jax.ShapeDtypeStruct((T, E), jnp.float32),
        ],
        interpret=interpret,
    )(x, ln_w.reshape(1, C), ln_b.reshape(1, C), Wg)

    topi_t, gates_t, key_t, cnt, aux, bal = pl.pallas_call(
        _route_kernel,
        out_shape=[
            jax.ShapeDtypeStruct((K, T), jnp.int32),
            jax.ShapeDtypeStruct((K, T), jnp.float32),
            jax.ShapeDtypeStruct((K, T), jnp.int32),
            jax.ShapeDtypeStruct((2, E), jnp.int32),
            jax.ShapeDtypeStruct((1, 1), jnp.float32),
            jax.ShapeDtypeStruct((1, 1), jnp.float32),
        ],
        interpret=interpret,
    )(logits)

    iota_t = jnp.arange(T, dtype=jnp.int32)
    ks, tok_s, g_s = jax.lax.sort(
        (key_t.reshape(TK), jnp.concatenate([iota_t, iota_t]),
         gates_t.reshape(TK)), num_keys=1)
    srow = ks % TK
    _, dest = jax.lax.sort((srow, jnp.arange(TK, dtype=jnp.int32)),
                           num_keys=1)
    dest2 = dest.reshape(T, K)

    counts = cnt[0]
    off = cnt[1]
    end = off + counts
    first_b = off // BLK
    nb = jnp.where(counts > 0, jnp.maximum(end - 1, 0) // BLK - first_b + 1, 0)
    cum_nb = jnp.cumsum(nb)
    slot_start = cum_nb - nb
    sidx = jnp.arange(NT, dtype=jnp.int32)
    te = jnp.searchsorted(cum_nb, sidx, side="right").astype(jnp.int32)
    tv = (sidx < cum_nb[-1]).astype(jnp.int32)
    te = jnp.minimum(te, E - 1)
    rb = first_b[te] + (sidx - slot_start[te])
    rb = jnp.where(tv > 0, rb, NB - 1).astype(jnp.int32)
    lo = jnp.maximum(rb * BLK, off[te]).astype(jnp.int32)
    hi = jnp.minimum((rb + 1) * BLK, end[te]).astype(jnp.int32)

    xg = jnp.take(xb, tok_s, axis=0)
    g_col = g_s.reshape(NB, BLK, 1)

    grid_spec = pltpu.PrefetchScalarGridSpec(
        num_scalar_prefetch=5,
        grid=(NT,),
        in_specs=[
            pl.BlockSpec((BLK, C), lambda s, te, rb, lo, hi, tv: (rb[s], 0)),
            pl.BlockSpec((1, BLK, 1), lambda s, te, rb, lo, hi, tv: (rb[s], 0, 0)),
            pl.BlockSpec((1, C, H), lambda s, te, rb, lo, hi, tv: (te[s], 0, 0)),
            pl.BlockSpec((1, 1, H), lambda s, te, rb, lo, hi, tv: (te[s], 0, 0)),
            pl.BlockSpec((1, H, H), lambda s, te, rb, lo, hi, tv: (te[s], 0, 0)),
            pl.BlockSpec((1, 1, H), lambda s, te, rb, lo, hi, tv: (te[s], 0, 0)),
        ],
        out_specs=pl.BlockSpec((BLK, H), lambda s, te, rb, lo, hi, tv: (rb[s], 0)),
    )
    y_s = pl.pallas_call(
        _expert_kernel,
        grid_spec=grid_spec,
        out_shape=jax.ShapeDtypeStruct((TK, H), jnp.float32),
        interpret=interpret,
    )(te, rb, lo, hi, tv, xg, g_col, W1, b1.reshape(E, 1, H), W2,
      b2.reshape(E, 1, H))

    out = (jnp.take(y_s, dest2[:, 0], axis=0)
           + jnp.take(y_s, dest2[:, 1], axis=0))

    results = out.reshape(Bb, S, H)
    id_experts = jnp.transpose(topi_t).reshape(Bb, S, K)
    return results, aux[0, 0], id_experts, bal[0, 0]


# speedup vs baseline: 1.8106x; 1.0699x over previous
import functools

import jax
import jax.numpy as jnp
from jax.experimental import pallas as pl
from jax.experimental.pallas import tpu as pltpu

E = 8
K = 2
BLK = 256
RBLK = 256
NEG = -1e30


def _ln_router_kernel(x_ref, lnw_ref, lnb_ref, wg_ref, xb_ref, logits_ref):
    x = x_ref[...]
    mu = jnp.mean(x, axis=-1, keepdims=True)
    var = jnp.mean((x - mu) ** 2, axis=-1, keepdims=True)
    xn = (x - mu) / jnp.sqrt(var + 1e-6) * lnw_ref[...] + lnb_ref[...]
    xb_ref[...] = xn.astype(jnp.bfloat16)
    logits_ref[...] = jnp.dot(xn, wg_ref[...],
                              preferred_element_type=jnp.float32)


def _route_kernel(logits_ref, topi_ref, gates_ref, key_ref, cnt_ref,
                  aux_ref, bal_ref):
    lt = jnp.transpose(logits_ref[...])
    T = lt.shape[1]
    TK = T * K
    iota_e = jax.lax.broadcasted_iota(jnp.int32, (E, T), 0)
    m1 = jnp.max(lt, axis=0, keepdims=True)
    i1 = jnp.min(jnp.where(lt == m1, iota_e, E), axis=0, keepdims=True)
    masked = jnp.where(iota_e == i1, NEG, lt)
    m2 = jnp.max(masked, axis=0, keepdims=True)
    i2 = jnp.min(jnp.where(masked == m2, iota_e, E), axis=0, keepdims=True)
    topi_ref[...] = jnp.concatenate([i1, i2], axis=0)
    g1 = 1.0 / (1.0 + jnp.exp(m2 - m1))
    gates_ref[...] = jnp.concatenate([g1, 1.0 - g1], axis=0)
    iota_t = jax.lax.broadcasted_iota(jnp.int32, (1, T), 1)
    key_ref[...] = jnp.concatenate(
        [i1 * TK + iota_t * K, i2 * TK + iota_t * K + 1], axis=0)
    both = ((i1 == iota_e).astype(jnp.int32)
            + (i2 == iota_e).astype(jnp.int32))
    counts = jnp.sum(both, axis=1, keepdims=True)
    acc = counts
    shift = 1
    while shift < E:
        acc = acc + jnp.concatenate(
            [jnp.zeros((shift, 1), jnp.int32), acc[:-shift]], axis=0)
        shift *= 2
    off = acc - counts
    cnt_ref[...] = jnp.concatenate(
        [jnp.transpose(counts), jnp.transpose(off)], axis=0)
    ex = jnp.exp(lt - m1)
    sumex = jnp.sum(ex, axis=0, keepdims=True)
    P = jnp.mean(ex / sumex, axis=1, keepdims=True)
    dens = counts.astype(jnp.float32) / T
    aux_ref[...] = (E * jnp.sum(dens * P)).reshape(1, 1)
    z = m1 + jnp.log(sumex)
    bal_ref[...] = jnp.mean(z * z).reshape(1, 1)


def _expert_kernel(te_ref, rb_ref, lo_ref, hi_ref, tv_ref,
                   xg_ref, g_ref, w1_ref, b1_ref, w2_ref, b2_ref, out_ref):
    s = pl.program_id(0)

    @pl.when(tv_ref[s] > 0)
    def _():
        e = te_ref[s]
        w1 = w1_ref[e].astype(jnp.bfloat16)
        h = jnp.dot(xg_ref[...], w1, preferred_element_type=jnp.float32)
        h = jax.nn.gelu(h + b1_ref[e])
        w2 = w2_ref[e].astype(jnp.bfloat16)
        y = jnp.dot(h.astype(jnp.bfloat16), w2,
                    preferred_element_type=jnp.float32)
        y = (y + b2_ref[e]) * g_ref[0]
        row = rb_ref[s] * BLK + jax.lax.broadcasted_iota(
            jnp.int32, (BLK, 1), 0)
        mask = (row >= lo_ref[s]) & (row < hi_ref[s])
        out_ref[...] = jnp.where(mask, y.astype(jnp.bfloat16), out_ref[...])


@functools.partial(jax.jit, static_argnames=("interpret",))
def kernel(x_img, ln_w, ln_b, Wg, W1, b1, W2, b2, interpret=False):
    Bb, S, C = x_img.shape
    T = Bb * S
    H = W2.shape[-1]
    TK = T * K
    NB = TK // BLK
    NT = NB + E - 1
    x = x_img.reshape(T, C)

    xb, logits = pl.pallas_call(
        _ln_router_kernel,
        grid=(T // RBLK,),
        in_specs=[
            pl.BlockSpec((RBLK, C), lambda i: (i, 0)),
            pl.BlockSpec((1, C), lambda i: (0, 0)),
            pl.BlockSpec((1, C), lambda i: (0, 0)),
            pl.BlockSpec((C, E), lambda i: (0, 0)),
        ],
        out_specs=[
            pl.BlockSpec((RBLK, C), lambda i: (i, 0)),
            pl.BlockSpec((RBLK, E), lambda i: (i, 0)),
        ],
        out_shape=[
            jax.ShapeDtypeStruct((T, C), jnp.bfloat16),
            jax.ShapeDtypeStruct((T, E), jnp.float32),
        ],
        interpret=interpret,
    )(x, ln_w.reshape(1, C), ln_b.reshape(1, C), Wg)

    topi_t, gates_t, key_t, cnt, aux, bal = pl.pallas_call(
        _route_kernel,
        out_shape=[
            jax.ShapeDtypeStruct((K, T), jnp.int32),
            jax.ShapeDtypeStruct((K, T), jnp.float32),
            jax.ShapeDtypeStruct((K, T), jnp.int32),
            jax.ShapeDtypeStruct((2, E), jnp.int32),
            jax.ShapeDtypeStruct((1, 1), jnp.float32),
            jax.ShapeDtypeStruct((1, 1), jnp.float32),
        ],
        interpret=interpret,
    )(logits)

    iota_t = jnp.arange(T, dtype=jnp.int32)
    ks, tok_s, g_s = jax.lax.sort(
        (key_t.reshape(TK), jnp.concatenate([iota_t, iota_t]),
         gates_t.reshape(TK)), num_keys=1)
    srow = ks % TK
    _, dest = jax.lax.sort((srow, jnp.arange(TK, dtype=jnp.int32)),
                           num_keys=1)
    dest2 = dest.reshape(T, K)

    counts = cnt[0]
    off = cnt[1]
    end = off + counts
    first_b = off // BLK
    nb = jnp.where(counts > 0, jnp.maximum(end - 1, 0) // BLK - first_b + 1, 0)
    cum_nb = jnp.cumsum(nb)
    slot_start = cum_nb - nb
    sidx = jnp.arange(NT, dtype=jnp.int32)
    te = jnp.searchsorted(cum_nb, sidx, side="right").astype(jnp.int32)
    tv = (sidx < cum_nb[-1]).astype(jnp.int32)
    te = jnp.minimum(te, E - 1)
    rb = first_b[te] + (sidx - slot_start[te])
    rb = jnp.where(tv > 0, rb, NB - 1).astype(jnp.int32)
    lo = jnp.maximum(rb * BLK, off[te]).astype(jnp.int32)
    hi = jnp.minimum((rb + 1) * BLK, end[te]).astype(jnp.int32)

    xg = jnp.take(xb, tok_s, axis=0)
    g_col = g_s.reshape(NB, BLK, 1)

    grid_spec = pltpu.PrefetchScalarGridSpec(
        num_scalar_prefetch=5,
        grid=(NT,),
        in_specs=[
            pl.BlockSpec((BLK, C), lambda s, te, rb, lo, hi, tv: (rb[s], 0)),
            pl.BlockSpec((1, BLK, 1), lambda s, te, rb, lo, hi, tv: (rb[s], 0, 0)),
            pl.BlockSpec((E, C, H), lambda s, te, rb, lo, hi, tv: (0, 0, 0)),
            pl.BlockSpec((E, 1, H), lambda s, te, rb, lo, hi, tv: (0, 0, 0)),
            pl.BlockSpec((E, H, H), lambda s, te, rb, lo, hi, tv: (0, 0, 0)),
            pl.BlockSpec((E, 1, H), lambda s, te, rb, lo, hi, tv: (0, 0, 0)),
        ],
        out_specs=pl.BlockSpec((BLK, H), lambda s, te, rb, lo, hi, tv: (rb[s], 0)),
    )
    y_s = pl.pallas_call(
        _expert_kernel,
        grid_spec=grid_spec,
        out_shape=jax.ShapeDtypeStruct((TK, H), jnp.bfloat16),
        interpret=interpret,
    )(te, rb, lo, hi, tv, xg, g_col, W1, b1.reshape(E, 1, H), W2,
      b2.reshape(E, 1, H))

    out = (jnp.take(y_s, dest2[:, 0], axis=0).astype(jnp.float32)
           + jnp.take(y_s, dest2[:, 1], axis=0).astype(jnp.float32))

    results = out.reshape(Bb, S, H)
    id_experts = jnp.transpose(topi_t).reshape(Bb, S, K)
    return results, aux[0, 0], id_experts, bal[0, 0]


# speedup vs baseline: 1.8349x; 1.0134x over previous
import functools

import jax
import jax.numpy as jnp
from jax.experimental import pallas as pl
from jax.experimental.pallas import tpu as pltpu

E = 8
K = 2
BLK = 512
RBLK = 256
NEG = -1e30


def _ln_router_kernel(x_ref, lnw_ref, lnb_ref, wg_ref, xb_ref, logits_ref):
    x = x_ref[...]
    mu = jnp.mean(x, axis=-1, keepdims=True)
    var = jnp.mean((x - mu) ** 2, axis=-1, keepdims=True)
    xn = (x - mu) / jnp.sqrt(var + 1e-6) * lnw_ref[...] + lnb_ref[...]
    xb_ref[...] = xn.astype(jnp.bfloat16)
    logits_ref[...] = jnp.dot(xn, wg_ref[...],
                              preferred_element_type=jnp.float32)


def _route_kernel(logits_ref, topi_ref, gates_ref, key_ref, cnt_ref,
                  aux_ref, bal_ref):
    lt = jnp.transpose(logits_ref[...])
    T = lt.shape[1]
    TK = T * K
    iota_e = jax.lax.broadcasted_iota(jnp.int32, (E, T), 0)
    m1 = jnp.max(lt, axis=0, keepdims=True)
    i1 = jnp.min(jnp.where(lt == m1, iota_e, E), axis=0, keepdims=True)
    masked = jnp.where(iota_e == i1, NEG, lt)
    m2 = jnp.max(masked, axis=0, keepdims=True)
    i2 = jnp.min(jnp.where(masked == m2, iota_e, E), axis=0, keepdims=True)
    topi_ref[...] = jnp.concatenate([i1, i2], axis=0)
    g1 = 1.0 / (1.0 + jnp.exp(m2 - m1))
    gates_ref[...] = jnp.concatenate([g1, 1.0 - g1], axis=0)
    iota_t = jax.lax.broadcasted_iota(jnp.int32, (1, T), 1)
    key_ref[...] = jnp.concatenate(
        [i1 * TK + iota_t * K, i2 * TK + iota_t * K + 1], axis=0)
    both = ((i1 == iota_e).astype(jnp.int32)
            + (i2 == iota_e).astype(jnp.int32))
    counts = jnp.sum(both, axis=1, keepdims=True)
    acc = counts
    shift = 1
    while shift < E:
        acc = acc + jnp.concatenate(
            [jnp.zeros((shift, 1), jnp.int32), acc[:-shift]], axis=0)
        shift *= 2
    off = acc - counts
    cnt_ref[...] = jnp.concatenate(
        [jnp.transpose(counts), jnp.transpose(off)], axis=0)
    ex = jnp.exp(lt - m1)
    sumex = jnp.sum(ex, axis=0, keepdims=True)
    P = jnp.mean(ex / sumex, axis=1, keepdims=True)
    dens = counts.astype(jnp.float32) / T
    aux_ref[...] = (E * jnp.sum(dens * P)).reshape(1, 1)
    z = m1 + jnp.log(sumex)
    bal_ref[...] = jnp.mean(z * z).reshape(1, 1)


def _expert_kernel(te_ref, rb_ref, lo_ref, hi_ref, tv_ref,
                   xg_ref, g_ref, w1_ref, b1_ref, w2_ref, b2_ref, out_ref):
    s = pl.program_id(0)

    @pl.when(tv_ref[s] > 0)
    def _():
        e = te_ref[s]
        w1 = w1_ref[e].astype(jnp.bfloat16)
        h = jnp.dot(xg_ref[...], w1, preferred_element_type=jnp.float32)
        h = jax.nn.gelu(h + b1_ref[e])
        w2 = w2_ref[e].astype(jnp.bfloat16)
        y = jnp.dot(h.astype(jnp.bfloat16), w2,
                    preferred_element_type=jnp.float32)
        y = (y + b2_ref[e]) * g_ref[0]
        row = rb_ref[s] * BLK + jax.lax.broadcasted_iota(
            jnp.int32, (BLK, 1), 0)
        mask = (row >= lo_ref[s]) & (row < hi_ref[s])
        out_ref[...] = jnp.where(mask, y.astype(jnp.bfloat16), out_ref[...])


@functools.partial(jax.jit, static_argnames=("interpret",))
def kernel(x_img, ln_w, ln_b, Wg, W1, b1, W2, b2, interpret=False):
    Bb, S, C = x_img.shape
    T = Bb * S
    H = W2.shape[-1]
    TK = T * K
    NB = TK // BLK
    NT = NB + E - 1
    x = x_img.reshape(T, C)

    xb, logits = pl.pallas_call(
        _ln_router_kernel,
        grid=(T // RBLK,),
        in_specs=[
            pl.BlockSpec((RBLK, C), lambda i: (i, 0)),
            pl.BlockSpec((1, C), lambda i: (0, 0)),
            pl.BlockSpec((1, C), lambda i: (0, 0)),
            pl.BlockSpec((C, E), lambda i: (0, 0)),
        ],
        out_specs=[
            pl.BlockSpec((RBLK, C), lambda i: (i, 0)),
            pl.BlockSpec((RBLK, E), lambda i: (i, 0)),
        ],
        out_shape=[
            jax.ShapeDtypeStruct((T, C), jnp.bfloat16),
            jax.ShapeDtypeStruct((T, E), jnp.float32),
        ],
        interpret=interpret,
    )(x, ln_w.reshape(1, C), ln_b.reshape(1, C), Wg)

    topi_t, gates_t, key_t, cnt, aux, bal = pl.pallas_call(
        _route_kernel,
        out_shape=[
            jax.ShapeDtypeStruct((K, T), jnp.int32),
            jax.ShapeDtypeStruct((K, T), jnp.float32),
            jax.ShapeDtypeStruct((K, T), jnp.int32),
            jax.ShapeDtypeStruct((2, E), jnp.int32),
            jax.ShapeDtypeStruct((1, 1), jnp.float32),
            jax.ShapeDtypeStruct((1, 1), jnp.float32),
        ],
        interpret=interpret,
    )(logits)

    iota_t = jnp.arange(T, dtype=jnp.int32)
    ks, tok_s, g_s = jax.lax.sort(
        (key_t.reshape(TK), jnp.concatenate([iota_t, iota_t]),
         gates_t.reshape(TK)), num_keys=1)
    srow = ks % TK
    _, dest = jax.lax.sort((srow, jnp.arange(TK, dtype=jnp.int32)),
                           num_keys=1)
    dest2 = dest.reshape(T, K)

    counts = cnt[0]
    off = cnt[1]
    end = off + counts
    first_b = off // BLK
    nb = jnp.where(counts > 0, jnp.maximum(end - 1, 0) // BLK - first_b + 1, 0)
    cum_nb = jnp.cumsum(nb)
    slot_start = cum_nb - nb
    sidx = jnp.arange(NT, dtype=jnp.int32)
    te = jnp.searchsorted(cum_nb, sidx, side="right").astype(jnp.int32)
    tv = (sidx < cum_nb[-1]).astype(jnp.int32)
    te = jnp.minimum(te, E - 1)
    rb = first_b[te] + (sidx - slot_start[te])
    rb = jnp.where(tv > 0, rb, NB - 1).astype(jnp.int32)
    lo = jnp.maximum(rb * BLK, off[te]).astype(jnp.int32)
    hi = jnp.minimum((rb + 1) * BLK, end[te]).astype(jnp.int32)

    xg = jnp.take(xb, tok_s, axis=0)
    g_col = g_s.reshape(NB, BLK, 1)

    grid_spec = pltpu.PrefetchScalarGridSpec(
        num_scalar_prefetch=5,
        grid=(NT,),
        in_specs=[
            pl.BlockSpec((BLK, C), lambda s, te, rb, lo, hi, tv: (rb[s], 0)),
            pl.BlockSpec((1, BLK, 1), lambda s, te, rb, lo, hi, tv: (rb[s], 0, 0)),
            pl.BlockSpec((E, C, H), lambda s, te, rb, lo, hi, tv: (0, 0, 0)),
            pl.BlockSpec((E, 1, H), lambda s, te, rb, lo, hi, tv: (0, 0, 0)),
            pl.BlockSpec((E, H, H), lambda s, te, rb, lo, hi, tv: (0, 0, 0)),
            pl.BlockSpec((E, 1, H), lambda s, te, rb, lo, hi, tv: (0, 0, 0)),
        ],
        out_specs=pl.BlockSpec((BLK, H), lambda s, te, rb, lo, hi, tv: (rb[s], 0)),
    )
    y_s = pl.pallas_call(
        _expert_kernel,
        grid_spec=grid_spec,
        out_shape=jax.ShapeDtypeStruct((TK, H), jnp.bfloat16),
        interpret=interpret,
    )(te, rb, lo, hi, tv, xg, g_col, W1, b1.reshape(E, 1, H), W2,
      b2.reshape(E, 1, H))

    out = (jnp.take(y_s, dest2[:, 0], axis=0).astype(jnp.float32)
           + jnp.take(y_s, dest2[:, 1], axis=0).astype(jnp.float32))

    results = out.reshape(Bb, S, H)
    id_experts = jnp.transpose(topi_t).reshape(Bb, S, K)
    return results, aux[0, 0], id_experts, bal[0, 0]


# speedup vs baseline: 2.9288x; 1.5962x over previous
import functools

import jax
import jax.numpy as jnp
from jax.experimental import pallas as pl
from jax.experimental.pallas import tpu as pltpu

E = 8
K = 2
RBLK = 256
NEG = -1e30


def _ln_router_kernel(x_ref, lnw_ref, lnb_ref, wg_ref, xb_ref, logits_ref):
    x = x_ref[...]
    mu = jnp.mean(x, axis=-1, keepdims=True)
    var = jnp.mean((x - mu) ** 2, axis=-1, keepdims=True)
    xn = (x - mu) / jnp.sqrt(var + 1e-6) * lnw_ref[...] + lnb_ref[...]
    xb_ref[...] = xn.astype(jnp.bfloat16)
    logits_ref[...] = jnp.dot(xn, wg_ref[...],
                              preferred_element_type=jnp.float32)


def _route_kernel(logits_ref, topi_ref, cw_ref, aux_ref, bal_ref):
    lt = jnp.transpose(logits_ref[...])
    T = lt.shape[1]
    iota_e = jax.lax.broadcasted_iota(jnp.int32, (E, T), 0)
    m1 = jnp.max(lt, axis=0, keepdims=True)
    i1 = jnp.min(jnp.where(lt == m1, iota_e, E), axis=0, keepdims=True)
    masked = jnp.where(iota_e == i1, NEG, lt)
    m2 = jnp.max(masked, axis=0, keepdims=True)
    i2 = jnp.min(jnp.where(masked == m2, iota_e, E), axis=0, keepdims=True)
    topi_ref[...] = jnp.concatenate([i1, i2], axis=0)
    g1 = 1.0 / (1.0 + jnp.exp(m2 - m1))
    oh1 = (i1 == iota_e).astype(jnp.float32)
    oh2 = (i2 == iota_e).astype(jnp.float32)
    cw_ref[...] = oh1 * g1 + oh2 * (1.0 - g1)
    ex = jnp.exp(lt - m1)
    sumex = jnp.sum(ex, axis=0, keepdims=True)
    P = jnp.mean(ex / sumex, axis=1, keepdims=True)
    dens = jnp.mean(oh1 + oh2, axis=1, keepdims=True)
    aux_ref[...] = (E * jnp.sum(dens * P)).reshape(1, 1)
    z = m1 + jnp.log(sumex)
    bal_ref[...] = jnp.mean(z * z).reshape(1, 1)


def _dense_expert_kernel(xb_ref, cw_ref, w1_ref, b1_ref, w2_ref, b2_ref,
                         out_ref, acc_ref):
    e = pl.program_id(0)
    w1 = w1_ref[0].astype(jnp.bfloat16)
    h = jnp.dot(xb_ref[...], w1, preferred_element_type=jnp.float32)
    h = jax.nn.gelu(h + b1_ref[0])
    w2 = w2_ref[0].astype(jnp.bfloat16)
    y = jnp.dot(h.astype(jnp.bfloat16), w2, preferred_element_type=jnp.float32)
    contrib = (y + b2_ref[0]) * cw_ref[0]

    @pl.when(e == 0)
    def _():
        acc_ref[...] = contrib

    @pl.when(e > 0)
    def _():
        acc_ref[...] = acc_ref[...] + contrib

    @pl.when(e == E - 1)
    def _():
        out_ref[...] = acc_ref[...]


@functools.partial(jax.jit, static_argnames=("interpret",))
def kernel(x_img, ln_w, ln_b, Wg, W1, b1, W2, b2, interpret=False):
    Bb, S, C = x_img.shape
    T = Bb * S
    H = W2.shape[-1]
    x = x_img.reshape(T, C)

    xb, logits = pl.pallas_call(
        _ln_router_kernel,
        grid=(T // RBLK,),
        in_specs=[
            pl.BlockSpec((RBLK, C), lambda i: (i, 0)),
            pl.BlockSpec((1, C), lambda i: (0, 0)),
            pl.BlockSpec((1, C), lambda i: (0, 0)),
            pl.BlockSpec((C, E), lambda i: (0, 0)),
        ],
        out_specs=[
            pl.BlockSpec((RBLK, C), lambda i: (i, 0)),
            pl.BlockSpec((RBLK, E), lambda i: (i, 0)),
        ],
        out_shape=[
            jax.ShapeDtypeStruct((T, C), jnp.bfloat16),
            jax.ShapeDtypeStruct((T, E), jnp.float32),
        ],
        interpret=interpret,
    )(x, ln_w.reshape(1, C), ln_b.reshape(1, C), Wg)

    topi_t, cw, aux, bal = pl.pallas_call(
        _route_kernel,
        out_shape=[
            jax.ShapeDtypeStruct((K, T), jnp.int32),
            jax.ShapeDtypeStruct((E, T), jnp.float32),
            jax.ShapeDtypeStruct((1, 1), jnp.float32),
            jax.ShapeDtypeStruct((1, 1), jnp.float32),
        ],
        interpret=interpret,
    )(logits)

    out = pl.pallas_call(
        _dense_expert_kernel,
        grid=(E,),
        in_specs=[
            pl.BlockSpec((T, C), lambda e: (0, 0)),
            pl.BlockSpec((1, T, 1), lambda e: (e, 0, 0)),
            pl.BlockSpec((1, C, H), lambda e: (e, 0, 0)),
            pl.BlockSpec((1, 1, H), lambda e: (e, 0, 0)),
            pl.BlockSpec((1, H, H), lambda e: (e, 0, 0)),
            pl.BlockSpec((1, 1, H), lambda e: (e, 0, 0)),
        ],
        out_specs=pl.BlockSpec((T, H), lambda e: (0, 0)),
        out_shape=jax.ShapeDtypeStruct((T, H), jnp.float32),
        scratch_shapes=[pltpu.VMEM((T, H), jnp.float32)],
        interpret=interpret,
    )(xb, cw.reshape(E, T, 1), W1, b1.reshape(E, 1, H), W2,
      b2.reshape(E, 1, H))

    results = out.reshape(Bb, S, H)
    id_experts = jnp.transpose(topi_t).reshape(Bb, S, K)
    return results, aux[0, 0], id_experts, bal[0, 0]


# speedup vs baseline: 3.3966x; 1.1597x over previous
import functools

import jax
import jax.numpy as jnp
from jax.experimental import pallas as pl
from jax.experimental.pallas import tpu as pltpu

E = 8
K = 2
RBLK = 512
NEG = -1e30


def _ln_router_kernel(x_ref, lnw_ref, lnb_ref, wg_ref, xb_ref, logits_ref):
    x = x_ref[0]
    mu = jnp.mean(x, axis=-1, keepdims=True)
    var = jnp.mean((x - mu) ** 2, axis=-1, keepdims=True)
    xn = (x - mu) / jnp.sqrt(var + 1e-6) * lnw_ref[...] + lnb_ref[...]
    xb_ref[...] = xn.astype(jnp.bfloat16)
    logits_ref[...] = jnp.dot(xn, wg_ref[...],
                              preferred_element_type=jnp.float32)


def _route_kernel(logits_ref, topi_ref, cw_ref, aux_ref, bal_ref):
    lt = jnp.transpose(logits_ref[...])
    T = lt.shape[1]
    iota_e = jax.lax.broadcasted_iota(jnp.int32, (E, T), 0)
    m1 = jnp.max(lt, axis=0, keepdims=True)
    i1 = jnp.min(jnp.where(lt == m1, iota_e, E), axis=0, keepdims=True)
    masked = jnp.where(iota_e == i1, NEG, lt)
    m2 = jnp.max(masked, axis=0, keepdims=True)
    i2 = jnp.min(jnp.where(masked == m2, iota_e, E), axis=0, keepdims=True)
    topi_ref[...] = jnp.concatenate([i1, i2], axis=0)
    g1 = 1.0 / (1.0 + jnp.exp(m2 - m1))
    oh1 = (i1 == iota_e).astype(jnp.float32)
    oh2 = (i2 == iota_e).astype(jnp.float32)
    cw_ref[...] = (oh1 * g1 + oh2 * (1.0 - g1)).reshape(E, 1, T)
    ex = jnp.exp(lt - m1)
    sumex = jnp.sum(ex, axis=0, keepdims=True)
    P = jnp.mean(ex / sumex, axis=1, keepdims=True)
    dens = jnp.mean(oh1 + oh2, axis=1, keepdims=True)
    aux_ref[...] = (E * jnp.sum(dens * P)).reshape(1, 1)
    z = m1 + jnp.log(sumex)
    bal_ref[...] = jnp.mean(z * z).reshape(1, 1)


def _dense_expert_kernel(xb_ref, cw_ref, w1_ref, b1_ref, w2_ref, b2_ref,
                         out_ref, acc_ref):
    e = pl.program_id(0)
    w1 = w1_ref[0].astype(jnp.bfloat16)
    h = jnp.dot(xb_ref[...], w1, preferred_element_type=jnp.float32)
    h = jax.nn.gelu(h + b1_ref[pl.ds(e, 1)])
    w2 = w2_ref[0].astype(jnp.bfloat16)
    y = jnp.dot(h.astype(jnp.bfloat16), w2, preferred_element_type=jnp.float32)
    cw_col = jnp.transpose(cw_ref[0])
    contrib = (y + b2_ref[pl.ds(e, 1)]) * cw_col

    @pl.when(e == 0)
    def _():
        acc_ref[...] = contrib

    @pl.when(e > 0)
    def _():
        acc_ref[...] = acc_ref[...] + contrib

    @pl.when(e == E - 1)
    def _():
        out_ref[...] = acc_ref[...]


@functools.partial(jax.jit, static_argnames=("interpret",))
def kernel(x_img, ln_w, ln_b, Wg, W1, b1, W2, b2, interpret=False):
    Bb, S, C = x_img.shape
    T = Bb * S
    H = W2.shape[-1]
    xb, logits = pl.pallas_call(
        _ln_router_kernel,
        grid=(T // RBLK,),
        in_specs=[
            pl.BlockSpec((1, RBLK, C), lambda i: (0, i, 0)),
            pl.BlockSpec((1, C), lambda i: (0, 0)),
            pl.BlockSpec((1, C), lambda i: (0, 0)),
            pl.BlockSpec((C, E), lambda i: (0, 0)),
        ],
        out_specs=[
            pl.BlockSpec((RBLK, C), lambda i: (i, 0)),
            pl.BlockSpec((RBLK, E), lambda i: (i, 0)),
        ],
        out_shape=[
            jax.ShapeDtypeStruct((T, C), jnp.bfloat16),
            jax.ShapeDtypeStruct((T, E), jnp.float32),
        ],
        interpret=interpret,
    )(x_img, ln_w.reshape(1, C), ln_b.reshape(1, C), Wg)

    topi_t, cw, aux, bal = pl.pallas_call(
        _route_kernel,
        out_shape=[
            jax.ShapeDtypeStruct((K, T), jnp.int32),
            jax.ShapeDtypeStruct((E, 1, T), jnp.float32),
            jax.ShapeDtypeStruct((1, 1), jnp.float32),
            jax.ShapeDtypeStruct((1, 1), jnp.float32),
        ],
        interpret=interpret,
    )(logits)

    out = pl.pallas_call(
        _dense_expert_kernel,
        grid=(E,),
        in_specs=[
            pl.BlockSpec((T, C), lambda e: (0, 0)),
            pl.BlockSpec((1, 1, T), lambda e: (e, 0, 0)),
            pl.BlockSpec((1, C, H), lambda e: (e, 0, 0)),
            pl.BlockSpec((E, H), lambda e: (0, 0)),
            pl.BlockSpec((1, H, H), lambda e: (e, 0, 0)),
            pl.BlockSpec((E, H), lambda e: (0, 0)),
        ],
        out_specs=pl.BlockSpec((T, H), lambda e: (0, 0)),
        out_shape=jax.ShapeDtypeStruct((T, H), jnp.float32),
        scratch_shapes=[pltpu.VMEM((T, H), jnp.float32)],
        interpret=interpret,
    )(xb, cw, W1, b1, W2, b2)

    results = out.reshape(Bb, S, H)
    id_experts = jnp.transpose(topi_t).reshape(Bb, S, K)
    return results, aux[0, 0], id_experts, bal[0, 0]


# speedup vs baseline: 3.8798x; 1.1423x over previous
import functools

import jax
import jax.numpy as jnp
from jax.experimental import pallas as pl
from jax.experimental.pallas import tpu as pltpu

E = 8
K = 2
RBLK = 512
NEG = -1e30


def _ln_router_kernel(x_ref, lnw_ref, lnb_ref, wg_ref, xb_ref, logits_ref):
    x = x_ref[0]
    mu = jnp.mean(x, axis=-1, keepdims=True)
    var = jnp.mean((x - mu) ** 2, axis=-1, keepdims=True)
    xn = (x - mu) / jnp.sqrt(var + 1e-6) * lnw_ref[...] + lnb_ref[...]
    xb_ref[...] = xn.astype(jnp.bfloat16)
    logits_ref[...] = jnp.dot(xn, wg_ref[...],
                              preferred_element_type=jnp.float32)


def _route_kernel(logits_ref, topi_ref, cw_ref, aux_ref, bal_ref):
    lt = jnp.transpose(logits_ref[...])
    T = lt.shape[1]
    iota_e = jax.lax.broadcasted_iota(jnp.int32, (E, T), 0)
    m1 = jnp.max(lt, axis=0, keepdims=True)
    i1 = jnp.min(jnp.where(lt == m1, iota_e, E), axis=0, keepdims=True)
    masked = jnp.where(iota_e == i1, NEG, lt)
    m2 = jnp.max(masked, axis=0, keepdims=True)
    i2 = jnp.min(jnp.where(masked == m2, iota_e, E), axis=0, keepdims=True)
    topi_ref[...] = jnp.concatenate([i1, i2], axis=0)
    g1 = 1.0 / (1.0 + jnp.exp(m2 - m1))
    oh1 = (i1 == iota_e).astype(jnp.float32)
    oh2 = (i2 == iota_e).astype(jnp.float32)
    cw_ref[...] = (oh1 * g1 + oh2 * (1.0 - g1)).reshape(E, 1, T)
    ex = jnp.exp(lt - m1)
    sumex = jnp.sum(ex, axis=0, keepdims=True)
    P = jnp.mean(ex / sumex, axis=1, keepdims=True)
    dens = jnp.mean(oh1 + oh2, axis=1, keepdims=True)
    aux_ref[...] = (E * jnp.sum(dens * P)).reshape(1, 1)
    z = m1 + jnp.log(sumex)
    bal_ref[...] = jnp.mean(z * z).reshape(1, 1)


def _dense_expert_kernel(xb_ref, cw_ref, w1_ref, b1_ref, w2_ref, b2_ref,
                         out_ref, acc_ref):
    e = pl.program_id(0)
    w1 = w1_ref[0].astype(jnp.bfloat16)
    h = jnp.dot(xb_ref[...], w1, preferred_element_type=jnp.float32)
    h = jax.nn.gelu((h + b1_ref[pl.ds(e, 1)]).astype(jnp.bfloat16))
    w2 = w2_ref[0].astype(jnp.bfloat16)
    y = jnp.dot(h, w2, preferred_element_type=jnp.float32)
    cw_col = jnp.transpose(cw_ref[0])
    contrib = (y + b2_ref[pl.ds(e, 1)]) * cw_col

    @pl.when(e == 0)
    def _():
        acc_ref[...] = contrib

    @pl.when(e > 0)
    def _():
        acc_ref[...] = acc_ref[...] + contrib

    @pl.when(e == E - 1)
    def _():
        out_ref[...] = acc_ref[...]


@functools.partial(jax.jit, static_argnames=("interpret",))
def kernel(x_img, ln_w, ln_b, Wg, W1, b1, W2, b2, interpret=False):
    Bb, S, C = x_img.shape
    T = Bb * S
    H = W2.shape[-1]
    xb, logits = pl.pallas_call(
        _ln_router_kernel,
        grid=(T // RBLK,),
        in_specs=[
            pl.BlockSpec((1, RBLK, C), lambda i: (0, i, 0)),
            pl.BlockSpec((C,), lambda i: (0,)),
            pl.BlockSpec((C,), lambda i: (0,)),
            pl.BlockSpec((C, E), lambda i: (0, 0)),
        ],
        out_specs=[
            pl.BlockSpec((RBLK, C), lambda i: (i, 0)),
            pl.BlockSpec((RBLK, E), lambda i: (i, 0)),
        ],
        out_shape=[
            jax.ShapeDtypeStruct((T, C), jnp.bfloat16),
            jax.ShapeDtypeStruct((T, E), jnp.float32),
        ],
        interpret=interpret,
    )(x_img, ln_w, ln_b, Wg)

    topi_t, cw, aux, bal = pl.pallas_call(
        _route_kernel,
        out_shape=[
            jax.ShapeDtypeStruct((K, T), jnp.int32),
            jax.ShapeDtypeStruct((E, 1, T), jnp.float32),
            jax.ShapeDtypeStruct((1, 1), jnp.float32),
            jax.ShapeDtypeStruct((1, 1), jnp.float32),
        ],
        interpret=interpret,
    )(logits)

    out = pl.pallas_call(
        _dense_expert_kernel,
        grid=(E,),
        in_specs=[
            pl.BlockSpec((T, C), lambda e: (0, 0)),
            pl.BlockSpec((1, 1, T), lambda e: (e, 0, 0)),
            pl.BlockSpec((1, C, H), lambda e: (e, 0, 0)),
            pl.BlockSpec((E, H), lambda e: (0, 0)),
            pl.BlockSpec((1, H, H), lambda e: (e, 0, 0)),
            pl.BlockSpec((E, H), lambda e: (0, 0)),
        ],
        out_specs=pl.BlockSpec((T, H), lambda e: (0, 0)),
        out_shape=jax.ShapeDtypeStruct((T, H), jnp.float32),
        scratch_shapes=[pltpu.VMEM((T, H), jnp.float32)],
        interpret=interpret,
    )(xb, cw, W1, b1, W2, b2)

    results = out.reshape(Bb, S, H)
    id_experts = jnp.transpose(topi_t).reshape(Bb, S, K)
    return results, aux[0, 0], id_experts, bal[0, 0]
